# Initial kernel scaffold; baseline (speedup 1.0000x reference)
import jax
import jax.numpy as jnp
from jax.experimental import pallas as pl


def kernel(inpt, centers0, centers1):
    raise NotImplementedError("write your pallas kernel here")


# speedup vs baseline: 1.5224x; 1.5224x over previous
import jax
import jax.numpy as jnp
from jax.experimental import pallas as pl

_BM = 1024


def _labeller_body(x_ref, ct_ref, out_ref):
    x = x_ref[...]
    ct = ct_ref[...]
    b2 = jnp.sum(ct * ct, axis=0)
    a2 = jnp.sum(x * x, axis=1, keepdims=True)
    dots = jax.lax.dot_general(
        x, ct, (((1,), (0,)), ((), ())),
        preferred_element_type=jnp.float32,
        precision=jax.lax.Precision.DEFAULT,
    )
    d2 = a2 + b2[None, :] - 2.0 * dots
    out_ref[...] = jnp.argmin(d2, axis=1).astype(jnp.int32)


def kernel(inpt, centers0, centers1):
    B, T, C = inpt.shape
    M = B * T
    x2d = inpt.reshape(M, C)
    ct = centers1.T
    out = pl.pallas_call(
        _labeller_body,
        grid=(M // _BM,),
        in_specs=[
            pl.BlockSpec((_BM, 128), lambda i: (i, 1)),
            pl.BlockSpec((128, 1024), lambda i: (0, 0)),
        ],
        out_specs=pl.BlockSpec((_BM,), lambda i: (i,)),
        out_shape=jax.ShapeDtypeStruct((M,), jnp.int32),
    )(x2d, ct)
    return out.reshape(B, T)


# speedup vs baseline: 1.5297x; 1.0048x over previous
import jax
import jax.numpy as jnp
from jax.experimental import pallas as pl

_BM = 1024


def _labeller_body(x_ref, ct2_ref, b2_ref, out_ref):
    x = x_ref[...]
    ct2 = ct2_ref[...]
    b2 = b2_ref[...]
    a2 = jnp.sum(x * x, axis=1, keepdims=True)
    dots2 = jax.lax.dot_general(
        x, ct2, (((1,), (0,)), ((), ())),
        preferred_element_type=jnp.float32,
        precision=jax.lax.Precision.DEFAULT,
    )
    d2 = (a2 + b2) + dots2
    out_ref[...] = jnp.argmin(d2, axis=1).astype(jnp.int32)


def kernel(inpt, centers0, centers1):
    B, T, C = inpt.shape
    M = B * T
    x2d = inpt.reshape(M, C)
    ct2 = centers1.T * -2.0
    b2 = jnp.sum(centers1 * centers1, axis=1)[None]
    out = pl.pallas_call(
        _labeller_body,
        grid=(M // _BM,),
        in_specs=[
            pl.BlockSpec((_BM, 128), lambda i: (i, 1)),
            pl.BlockSpec((128, 1024), lambda i: (0, 0)),
            pl.BlockSpec((1, 1024), lambda i: (0, 0)),
        ],
        out_specs=pl.BlockSpec((_BM,), lambda i: (i,)),
        out_shape=jax.ShapeDtypeStruct((M,), jnp.int32),
    )(x2d, ct2, b2)
    return out.reshape(B, T)
